# baseline (speedup 1.0000x reference)
import functools

import jax
import jax.numpy as jnp
from jax import lax
from jax.experimental import pallas as pl
from jax.experimental.pallas import tpu as pltpu
from jax.experimental.pallas import tpu_sc as plsc

N = 8192
D = 2048
F = 8192

BM = 512
BF = 1024
NPAD = N + 2 * BM
NB = NPAD // BM
KFF = F // BF

NW = 32
ROWS_PER_W = N // NW
CH = 16
NCHUNK = ROWS_PER_W // CH


def _cumsum_lanes(v):
    k = 1
    while k < N:
        shifted = jnp.concatenate(
            [jnp.zeros((1, k), v.dtype), v[:, : N - k]], axis=1
        )
        v = v + shifted
        k *= 2
    return v


def _route_body(ft_ref, pos_ref, meta_ref):
    ft = ft_ref[...]
    c_ed = _cumsum_lanes(ft)
    c_es = lax.broadcasted_iota(jnp.int32, (1, N), 1) + 1 - c_ed
    n_es = c_es[0, N - 1]
    n_ed = N - n_es
    n_es_pad = ((n_es + BM - 1) // BM) * BM
    pos = jnp.where(ft == 1, n_es_pad + c_ed - 1, c_es - 1)
    pos_ref[...] = pos
    nb_es = n_es_pad // BM
    nb_used = nb_es + (n_ed + BM - 1) // BM
    sel = lax.broadcasted_iota(jnp.int32, (1, 8), 1)
    meta_ref[...] = jnp.where(sel == 0, nb_es, nb_used)


def _route(ft2):
    return pl.pallas_call(
        _route_body,
        out_shape=(
            jax.ShapeDtypeStruct((1, N), jnp.int32),
            jax.ShapeDtypeStruct((1, 8), jnp.int32),
        ),
    )(ft2)


def _wid():
    return lax.axis_index("s") * 2 + lax.axis_index("c")


@functools.cache
def _sc_kernels():
    mesh = plsc.VectorSubcoreMesh(core_axis_name="c", subcore_axis_name="s")
    scratch = [
        pltpu.VMEM((NCHUNK, CH), jnp.int32),

        pltpu.VMEM((CH, D), jnp.float32),
        pltpu.VMEM((CH, D), jnp.float32),
        pltpu.SemaphoreType.DMA,
        pltpu.SemaphoreType.DMA,
        pltpu.SemaphoreType.DMA,
        pltpu.SemaphoreType.DMA,
    ]


    @functools.partial(
        pl.kernel,
        mesh=mesh,
        out_type=jax.ShapeDtypeStruct((NPAD, D), jnp.float32),
        scratch_types=scratch,
    )
    def sc_scatter(x_hbm, pos_hbm, out_hbm, idx_v, rows0, rows1, ls0, ls1, ss0, ss1):
        w = _wid()
        base = w * ROWS_PER_W
        pltpu.sync_copy(pos_hbm.at[w], idx_v)
        rows = (rows0, rows1)
        lsem = (ls0, ls1)
        ssem = (ss0, ss1)

        def load(i):
            return pltpu.make_async_copy(
                x_hbm.at[pl.ds(base + i * CH, CH), :], rows[i % 2], lsem[i % 2]
            )

        def scat(i):
            return pltpu.make_async_copy(
                rows[i % 2], out_hbm.at[idx_v.at[i]], ssem[i % 2]
            )

        load(0).start()
        for i in range(NCHUNK):
            if i >= 1:
                scat(i - 1).wait()
            if i + 1 < NCHUNK:
                load(i + 1).start()
            load(i).wait()
            scat(i).start()
        scat(NCHUNK - 1).wait()

    @functools.partial(
        pl.kernel,
        mesh=mesh,
        out_type=jax.ShapeDtypeStruct((N, D), jnp.float32),
        scratch_types=scratch,
    )
    def sc_gather(y_hbm, pos_hbm, out_hbm, idx_v, rows0, rows1, ls0, ls1, ss0, ss1):
        w = _wid()
        base = w * ROWS_PER_W
        pltpu.sync_copy(pos_hbm.at[w], idx_v)
        rows = (rows0, rows1)
        gsem = (ls0, ls1)
        wsem = (ss0, ss1)

        def gath(i):
            return pltpu.make_async_copy(
                y_hbm.at[idx_v.at[i]], rows[i % 2], gsem[i % 2]
            )

        def put(i):
            return pltpu.make_async_copy(
                rows[i % 2], out_hbm.at[pl.ds(base + i * CH, CH), :], wsem[i % 2]
            )

        gath(0).start()
        for i in range(NCHUNK):
            if i >= 1:
                put(i - 1).wait()
            if i + 1 < NCHUNK:
                gath(i + 1).start()
            gath(i).wait()
            put(i).start()
        put(NCHUNK - 1).wait()

    return sc_scatter, sc_gather


def _mlp_body(meta_ref, x_ref, w1_ref, b1_ref, w2_ref, b2_ref, out_ref):
    j2 = pl.program_id(0)
    k = pl.program_id(1)
    r = pl.program_id(2)
    g = 2 * j2 + r

    @pl.when(g < meta_ref[1])
    def _():
        rows = pl.ds(r * BM, BM)
        xb = x_ref[rows, :].astype(jnp.bfloat16)
        h = jnp.dot(xb, w1_ref[0], preferred_element_type=jnp.float32)
        h = jnp.maximum(h + b1_ref[0, 0], 0.0).astype(jnp.bfloat16)
        y = jnp.dot(h, w2_ref[0], preferred_element_type=jnp.float32)

        @pl.when(k == 0)
        def _():
            out_ref[rows, :] = y + b2_ref[0, 0]

        @pl.when(k > 0)
        def _():
            out_ref[rows, :] += y


def _expert(g, meta):
    return (g >= meta[0]).astype(jnp.int32)


def _mlp(meta, xs, w1s, b1s, w2s, b2s):
    grid_spec = pltpu.PrefetchScalarGridSpec(
        num_scalar_prefetch=1,
        grid=(NB // 2, KFF, 2),
        in_specs=[
            pl.BlockSpec((2 * BM, D), lambda j2, k, r, m: (j2, 0)),
            pl.BlockSpec((1, D, BF), lambda j2, k, r, m: (_expert(2 * j2 + r, m), 0, k)),
            pl.BlockSpec((1, 1, BF), lambda j2, k, r, m: (_expert(2 * j2 + r, m), 0, k)),
            pl.BlockSpec((1, BF, D), lambda j2, k, r, m: (_expert(2 * j2 + r, m), k, 0)),
            pl.BlockSpec((1, 1, D), lambda j2, k, r, m: (_expert(2 * j2 + r, m), 0, 0)),
        ],
        out_specs=pl.BlockSpec((2 * BM, D), lambda j2, k, r, m: (j2, 0)),
    )
    return pl.pallas_call(
        _mlp_body,
        grid_spec=grid_spec,
        out_shape=jax.ShapeDtypeStruct((NPAD, D), jnp.float32),
        compiler_params=pltpu.CompilerParams(
            dimension_semantics=("parallel", "arbitrary", "arbitrary"),
        ),
    )(meta, xs, w1s, b1s, w2s, b2s)


def kernel(x, frame_types, W1_ed, b1_ed, W2_ed, b2_ed, W1_es, b1_es, W2_es, b2_es):
    ft2 = frame_types.astype(jnp.int32).reshape(1, N)
    pos2, meta2 = _route(ft2)
    pos = pos2.reshape(NW, NCHUNK, CH)
    meta = meta2.reshape(8)

    w1s = jnp.stack([W1_es, W1_ed]).astype(jnp.bfloat16)
    w2s = jnp.stack([W2_es, W2_ed]).astype(jnp.bfloat16)
    b1s = jnp.stack([b1_es, b1_ed]).reshape(2, 1, F)
    b2s = jnp.stack([b2_es, b2_ed]).reshape(2, 1, D)

    sc_scatter, sc_gather = _sc_kernels()
    xs = sc_scatter(x, pos)

    ys = _mlp(meta, xs, w1s, b1s, w2s, b2s)

    out = sc_gather(ys, pos)
    return out.astype(x.dtype)


# speedup vs baseline: 1.0781x; 1.0781x over previous
import functools

import jax
import jax.numpy as jnp
from jax import lax
from jax.experimental import pallas as pl
from jax.experimental.pallas import tpu as pltpu
from jax.experimental.pallas import tpu_sc as plsc

N = 8192
D = 2048
F = 8192

BM = 512
BF = 2048
NPAD = N + 2 * BM
NB = NPAD // BM
KFF = F // BF

NW = 32
ROWS_PER_W = N // NW
CH = 16
NCHUNK = ROWS_PER_W // CH


def _cumsum_lanes(v):
    k = 1
    while k < N:
        shifted = jnp.concatenate(
            [jnp.zeros((1, k), v.dtype), v[:, : N - k]], axis=1
        )
        v = v + shifted
        k *= 2
    return v


def _route_body(ft_ref, pos_ref, meta_ref):
    ft = ft_ref[...]
    c_ed = _cumsum_lanes(ft)
    c_es = lax.broadcasted_iota(jnp.int32, (1, N), 1) + 1 - c_ed
    n_es = c_es[0, N - 1]
    n_ed = N - n_es
    n_es_pad = ((n_es + BM - 1) // BM) * BM
    pos = jnp.where(ft == 1, n_es_pad + c_ed - 1, c_es - 1)
    pos_ref[...] = pos
    nb_es = n_es_pad // BM
    nb_used = nb_es + (n_ed + BM - 1) // BM
    sel = lax.broadcasted_iota(jnp.int32, (1, 8), 1)
    meta_ref[...] = jnp.where(sel == 0, nb_es, nb_used)


def _route(ft2):
    return pl.pallas_call(
        _route_body,
        out_shape=(
            jax.ShapeDtypeStruct((1, N), jnp.int32),
            jax.ShapeDtypeStruct((1, 8), jnp.int32),
        ),
    )(ft2)


def _wid():
    return lax.axis_index("s") * 2 + lax.axis_index("c")


@functools.cache
def _sc_kernels():
    mesh = plsc.VectorSubcoreMesh(core_axis_name="c", subcore_axis_name="s")
    scratch = [
        pltpu.VMEM((NCHUNK, CH), jnp.int32),

        pltpu.VMEM((CH, D), jnp.float32),
        pltpu.VMEM((CH, D), jnp.float32),
        pltpu.SemaphoreType.DMA,
        pltpu.SemaphoreType.DMA,
        pltpu.SemaphoreType.DMA,
        pltpu.SemaphoreType.DMA,
    ]


    @functools.partial(
        pl.kernel,
        mesh=mesh,
        out_type=jax.ShapeDtypeStruct((NPAD, D), jnp.float32),
        scratch_types=scratch,
    )
    def sc_scatter(x_hbm, pos_hbm, out_hbm, idx_v, rows0, rows1, ls0, ls1, ss0, ss1):
        w = _wid()
        base = w * ROWS_PER_W
        pltpu.sync_copy(pos_hbm.at[w], idx_v)
        rows = (rows0, rows1)
        lsem = (ls0, ls1)
        ssem = (ss0, ss1)

        def load(i):
            return pltpu.make_async_copy(
                x_hbm.at[pl.ds(base + i * CH, CH), :], rows[i % 2], lsem[i % 2]
            )

        def scat(i):
            return pltpu.make_async_copy(
                rows[i % 2], out_hbm.at[idx_v.at[i]], ssem[i % 2]
            )

        load(0).start()
        for i in range(NCHUNK):
            if i >= 1:
                scat(i - 1).wait()
            if i + 1 < NCHUNK:
                load(i + 1).start()
            load(i).wait()
            scat(i).start()
        scat(NCHUNK - 1).wait()

    @functools.partial(
        pl.kernel,
        mesh=mesh,
        out_type=jax.ShapeDtypeStruct((N, D), jnp.float32),
        scratch_types=scratch,
    )
    def sc_gather(y_hbm, pos_hbm, out_hbm, idx_v, rows0, rows1, ls0, ls1, ss0, ss1):
        w = _wid()
        base = w * ROWS_PER_W
        pltpu.sync_copy(pos_hbm.at[w], idx_v)
        rows = (rows0, rows1)
        gsem = (ls0, ls1)
        wsem = (ss0, ss1)

        def gath(i):
            return pltpu.make_async_copy(
                y_hbm.at[idx_v.at[i]], rows[i % 2], gsem[i % 2]
            )

        def put(i):
            return pltpu.make_async_copy(
                rows[i % 2], out_hbm.at[pl.ds(base + i * CH, CH), :], wsem[i % 2]
            )

        gath(0).start()
        for i in range(NCHUNK):
            if i >= 1:
                put(i - 1).wait()
            if i + 1 < NCHUNK:
                gath(i + 1).start()
            gath(i).wait()
            put(i).start()
        put(NCHUNK - 1).wait()

    return sc_scatter, sc_gather


def _mlp_body(meta_ref, x_ref, w1_ref, b1_ref, w2_ref, b2_ref, out_ref):
    j = pl.program_id(0)
    k = pl.program_id(1)

    @pl.when(j < meta_ref[1])
    def _():
        xb = x_ref[...].astype(jnp.bfloat16)
        h = jnp.dot(xb, w1_ref[0], preferred_element_type=jnp.float32)
        h = jnp.maximum(h + b1_ref[0, 0], 0.0).astype(jnp.bfloat16)
        y = jnp.dot(h, w2_ref[0], preferred_element_type=jnp.float32)

        @pl.when(k == 0)
        def _():
            out_ref[...] = y + b2_ref[0, 0]

        @pl.when(k > 0)
        def _():
            out_ref[...] += y


def _expert(g, meta):
    return (g >= meta[0]).astype(jnp.int32)


def _mlp(meta, xs, w1s, b1s, w2s, b2s):
    grid_spec = pltpu.PrefetchScalarGridSpec(
        num_scalar_prefetch=1,
        grid=(NB, KFF),
        in_specs=[
            pl.BlockSpec((BM, D), lambda j, k, m: (j, 0)),
            pl.BlockSpec((1, D, BF), lambda j, k, m: (_expert(j, m), 0, k)),
            pl.BlockSpec((1, 1, BF), lambda j, k, m: (_expert(j, m), 0, k)),
            pl.BlockSpec((1, BF, D), lambda j, k, m: (_expert(j, m), k, 0)),
            pl.BlockSpec((1, 1, D), lambda j, k, m: (_expert(j, m), 0, 0)),
        ],
        out_specs=pl.BlockSpec((BM, D), lambda j, k, m: (j, 0)),
    )
    return pl.pallas_call(
        _mlp_body,
        grid_spec=grid_spec,
        out_shape=jax.ShapeDtypeStruct((NPAD, D), jnp.float32),
        compiler_params=pltpu.CompilerParams(
            dimension_semantics=("parallel", "arbitrary"),
        ),
    )(meta, xs, w1s, b1s, w2s, b2s)


def kernel(x, frame_types, W1_ed, b1_ed, W2_ed, b2_ed, W1_es, b1_es, W2_es, b2_es):
    ft2 = frame_types.astype(jnp.int32).reshape(1, N)
    pos2, meta2 = _route(ft2)
    pos = pos2.reshape(NW, NCHUNK, CH)
    meta = meta2.reshape(8)

    w1s = jnp.stack([W1_es.astype(jnp.bfloat16), W1_ed.astype(jnp.bfloat16)])
    w2s = jnp.stack([W2_es.astype(jnp.bfloat16), W2_ed.astype(jnp.bfloat16)])
    b1s = jnp.stack([b1_es, b1_ed]).reshape(2, 1, F)
    b2s = jnp.stack([b2_es, b2_ed]).reshape(2, 1, D)

    sc_scatter, sc_gather = _sc_kernels()
    xs = sc_scatter(x, pos)

    ys = _mlp(meta, xs, w1s, b1s, w2s, b2s)

    out = sc_gather(ys, pos)
    return out.astype(x.dtype)


# speedup vs baseline: 1.1783x; 1.0929x over previous
import functools

import jax
import jax.numpy as jnp
from jax import lax
from jax.experimental import pallas as pl
from jax.experimental.pallas import tpu as pltpu
from jax.experimental.pallas import tpu_sc as plsc

N = 8192
D = 2048
F = 8192

BM = 512
BF = 2048
NPAD = N + 2 * BM
NB = NPAD // BM
KFF = F // BF

NW = 32
ROWS_PER_W = N // NW
CH = 16
NCHUNK = ROWS_PER_W // CH


def _cumsum_lanes(v):
    k = 1
    while k < N:
        shifted = jnp.concatenate(
            [jnp.zeros((1, k), v.dtype), v[:, : N - k]], axis=1
        )
        v = v + shifted
        k *= 2
    return v


def _route_body(ft_ref, pos_ref, meta_ref):
    ft = ft_ref[...]
    c_ed = _cumsum_lanes(ft)
    c_es = lax.broadcasted_iota(jnp.int32, (1, N), 1) + 1 - c_ed
    n_es = c_es[0, N - 1]
    n_ed = N - n_es
    n_es_pad = ((n_es + BM - 1) // BM) * BM
    pos = jnp.where(ft == 1, n_es_pad + c_ed - 1, c_es - 1)
    pos_ref[...] = pos
    nb_es = n_es_pad // BM
    nb_used = nb_es + (n_ed + BM - 1) // BM
    sel = lax.broadcasted_iota(jnp.int32, (1, 8), 1)
    meta_ref[...] = jnp.where(sel == 0, nb_es, nb_used)


def _route(ft2):
    return pl.pallas_call(
        _route_body,
        out_shape=(
            jax.ShapeDtypeStruct((1, N), jnp.int32),
            jax.ShapeDtypeStruct((1, 8), jnp.int32),
        ),
    )(ft2)


def _wid():
    return lax.axis_index("s") * 2 + lax.axis_index("c")


@functools.cache
def _sc_kernels():
    mesh = plsc.VectorSubcoreMesh(core_axis_name="c", subcore_axis_name="s")
    scratch = [
        pltpu.VMEM((NCHUNK, CH), jnp.int32),

        pltpu.VMEM((CH, D), jnp.float32),
        pltpu.VMEM((CH, D), jnp.float32),
        pltpu.SemaphoreType.DMA,
        pltpu.SemaphoreType.DMA,
        pltpu.SemaphoreType.DMA,
        pltpu.SemaphoreType.DMA,
    ]


    @functools.partial(
        pl.kernel,
        mesh=mesh,
        out_type=jax.ShapeDtypeStruct((NPAD, D), jnp.float32),
        scratch_types=scratch,
    )
    def sc_scatter(x_hbm, pos_hbm, out_hbm, idx_v, rows0, rows1, ls0, ls1, ss0, ss1):
        w = _wid()
        base = w * ROWS_PER_W
        pltpu.sync_copy(pos_hbm.at[w], idx_v)
        rows = (rows0, rows1)
        lsem = (ls0, ls1)
        ssem = (ss0, ss1)

        def load(i):
            return pltpu.make_async_copy(
                x_hbm.at[pl.ds(base + i * CH, CH), :], rows[i % 2], lsem[i % 2]
            )

        def scat(i):
            return pltpu.make_async_copy(
                rows[i % 2], out_hbm.at[idx_v.at[i]], ssem[i % 2]
            )

        load(0).start()
        for i in range(NCHUNK):
            if i >= 1:
                scat(i - 1).wait()
            if i + 1 < NCHUNK:
                load(i + 1).start()
            load(i).wait()
            scat(i).start()
        scat(NCHUNK - 1).wait()

    @functools.partial(
        pl.kernel,
        mesh=mesh,
        out_type=jax.ShapeDtypeStruct((N, D), jnp.float32),
        scratch_types=scratch,
    )
    def sc_gather(y_hbm, pos_hbm, out_hbm, idx_v, rows0, rows1, ls0, ls1, ss0, ss1):
        w = _wid()
        base = w * ROWS_PER_W
        pltpu.sync_copy(pos_hbm.at[w], idx_v)
        rows = (rows0, rows1)
        gsem = (ls0, ls1)
        wsem = (ss0, ss1)

        def gath(i):
            return pltpu.make_async_copy(
                y_hbm.at[idx_v.at[i]], rows[i % 2], gsem[i % 2]
            )

        def put(i):
            return pltpu.make_async_copy(
                rows[i % 2], out_hbm.at[pl.ds(base + i * CH, CH), :], wsem[i % 2]
            )

        gath(0).start()
        for i in range(NCHUNK):
            if i >= 1:
                put(i - 1).wait()
            if i + 1 < NCHUNK:
                gath(i + 1).start()
            gath(i).wait()
            put(i).start()
        put(NCHUNK - 1).wait()

    return sc_scatter, sc_gather


_KS = 16


def _stack_body(w1e_ref, w1d_ref, w2e_ref, w2d_ref, w1s_ref, w2s_ref):
    e = pl.program_id(0)

    @pl.when(e == 0)
    def _():
        w1s_ref[0] = w1e_ref[...].astype(jnp.bfloat16)
        w2s_ref[0] = w2e_ref[...].astype(jnp.bfloat16)

    @pl.when(e == 1)
    def _():
        w1s_ref[0] = w1d_ref[...].astype(jnp.bfloat16)
        w2s_ref[0] = w2d_ref[...].astype(jnp.bfloat16)


def _stack_weights(W1_es, W1_ed, W2_es, W2_ed):
    cs = F // _KS
    return pl.pallas_call(
        _stack_body,
        grid=(2, _KS),
        in_specs=[
            pl.BlockSpec((D, cs), lambda e, c: (0, jnp.where(e == 0, c, _KS - 1))),
            pl.BlockSpec((D, cs), lambda e, c: (0, jnp.where(e == 1, c, 0))),
            pl.BlockSpec((cs, D), lambda e, c: (jnp.where(e == 0, c, _KS - 1), 0)),
            pl.BlockSpec((cs, D), lambda e, c: (jnp.where(e == 1, c, 0), 0)),
        ],
        out_specs=[
            pl.BlockSpec((1, D, cs), lambda e, c: (e, 0, c)),
            pl.BlockSpec((1, cs, D), lambda e, c: (e, c, 0)),
        ],
        out_shape=[
            jax.ShapeDtypeStruct((2, D, F), jnp.bfloat16),
            jax.ShapeDtypeStruct((2, F, D), jnp.bfloat16),
        ],
        compiler_params=pltpu.CompilerParams(
            dimension_semantics=("arbitrary", "arbitrary"),
        ),
    )(W1_es, W1_ed, W2_es, W2_ed)


def _mlp_body(meta_ref, x_ref, w1_ref, b1_ref, w2_ref, b2_ref, out_ref):
    j = pl.program_id(0)
    k = pl.program_id(1)

    @pl.when(j < meta_ref[1])
    def _():
        xb = x_ref[...].astype(jnp.bfloat16)
        h = jnp.dot(xb, w1_ref[0], preferred_element_type=jnp.float32)
        h = jnp.maximum(h + b1_ref[0, 0], 0.0).astype(jnp.bfloat16)
        y = jnp.dot(h, w2_ref[0], preferred_element_type=jnp.float32)

        @pl.when(k == 0)
        def _():
            out_ref[...] = y + b2_ref[0, 0]

        @pl.when(k > 0)
        def _():
            out_ref[...] += y


def _expert(g, meta):
    return (g >= meta[0]).astype(jnp.int32)


def _mlp(meta, xs, w1s, b1s, w2s, b2s):
    grid_spec = pltpu.PrefetchScalarGridSpec(
        num_scalar_prefetch=1,
        grid=(NB, KFF),
        in_specs=[
            pl.BlockSpec((BM, D), lambda j, k, m: (j, 0)),
            pl.BlockSpec((1, D, BF), lambda j, k, m: (_expert(j, m), 0, k)),
            pl.BlockSpec((1, 1, BF), lambda j, k, m: (_expert(j, m), 0, k)),
            pl.BlockSpec((1, BF, D), lambda j, k, m: (_expert(j, m), k, 0)),
            pl.BlockSpec((1, 1, D), lambda j, k, m: (_expert(j, m), 0, 0)),
        ],
        out_specs=pl.BlockSpec((BM, D), lambda j, k, m: (j, 0)),
    )
    return pl.pallas_call(
        _mlp_body,
        grid_spec=grid_spec,
        out_shape=jax.ShapeDtypeStruct((NPAD, D), jnp.float32),
        compiler_params=pltpu.CompilerParams(
            dimension_semantics=("parallel", "arbitrary"),
            vmem_limit_bytes=100 * 1024 * 1024,
        ),
    )(meta, xs, w1s, b1s, w2s, b2s)


def kernel(x, frame_types, W1_ed, b1_ed, W2_ed, b2_ed, W1_es, b1_es, W2_es, b2_es):
    ft2 = frame_types.astype(jnp.int32).reshape(1, N)
    pos2, meta2 = _route(ft2)
    pos = pos2.reshape(NW, NCHUNK, CH)
    meta = meta2.reshape(8)

    w1s, w2s = _stack_weights(W1_es, W1_ed, W2_es, W2_ed)
    b1s = jnp.stack([b1_es, b1_ed]).reshape(2, 1, F)
    b2s = jnp.stack([b2_es, b2_ed]).reshape(2, 1, D)

    sc_scatter, sc_gather = _sc_kernels()
    xs = sc_scatter(x, pos)

    ys = _mlp(meta, xs, w1s, b1s, w2s, b2s)

    out = sc_gather(ys, pos)
    return out.astype(x.dtype)
